# initial kernel scaffold (unmeasured)
import jax
import jax.numpy as jnp
from jax import lax
from jax.experimental import pallas as pl
from jax.experimental.pallas import tpu as pltpu

N_DEV = 4
N_EXPERTS = 16
E_LOCAL = N_EXPERTS // N_DEV


def kernel(x, router_W, route_idx, expert_W, shared_W):
    n_tok, d_model = x.shape
    d_ff = shared_W.shape[1]
    rows = n_tok // N_DEV

    def body(x_ref, rw_ref, idx_ref, ew_ref, sw_ref, out_ref,
             acc_ref, comm_ref, send_sems, recv_sems):
        my = lax.axis_index("i")
        left = lax.rem(my + N_DEV - 1, N_DEV)
        right = lax.rem(my + 1, N_DEV)

        barrier = pltpu.get_barrier_semaphore()
        for nbr in (left, right):
            pl.semaphore_signal(
                barrier, inc=1,
                device_id=(nbr,), device_id_type=pl.DeviceIdType.MESH,
            )
        pl.semaphore_wait(barrier, 2)

        x_f32 = x_ref[:, :]
        scores = jnp.dot(x_f32, rw_ref[:, :], preferred_element_type=jnp.float32)
        scores = scores - jnp.max(scores, axis=-1, keepdims=True)
        e_scores = jnp.exp(scores)
        probs = e_scores / jnp.sum(e_scores, axis=-1, keepdims=True)

        ridx = idx_ref[:, :]
        eidx = lax.broadcasted_iota(jnp.int32, probs.shape, 1)
        x_bf = x_f32.astype(jnp.bfloat16)

        partial = jnp.zeros((n_tok, d_ff), jnp.float32)
        for e_loc in range(E_LOCAL):
            e_glob = my * E_LOCAL + e_loc
            p_e = jnp.sum(jnp.where(eidx == e_glob, probs, 0.0),
                          axis=1, keepdims=True)
            w = jnp.where(ridx == e_glob, p_e, 0.0)
            y = jnp.dot(x_bf, ew_ref[e_loc].astype(jnp.bfloat16),
                        preferred_element_type=jnp.float32)
            partial = partial + w * y
        acc_ref[:, :] = partial.astype(jnp.bfloat16)

        for s in range(N_DEV - 1):
            send_idx = lax.rem(my - (s + 1) + 2 * N_DEV, N_DEV)
            recv_idx = lax.rem(my - (s + 2) + 2 * N_DEV, N_DEV)
            rdma = pltpu.make_async_remote_copy(
                src_ref=acc_ref.at[pl.ds(send_idx * rows, rows)],
                dst_ref=comm_ref.at[s],
                send_sem=send_sems.at[s],
                recv_sem=recv_sems.at[s],
                device_id=(right,),
                device_id_type=pl.DeviceIdType.MESH,
            )
            rdma.start()
            rdma.wait()
            acc_ref[pl.ds(recv_idx * rows, rows)] = (
                acc_ref[pl.ds(recv_idx * rows, rows)] + comm_ref[s]
            )

        x_mine = lax.dynamic_slice(x_bf, (my * rows, 0), (rows, d_model))
        shared = jnp.dot(x_mine, sw_ref[:, :].astype(jnp.bfloat16),
                         preferred_element_type=jnp.float32)
        out_ref[:, :] = shared + acc_ref[pl.ds(my * rows, rows)].astype(jnp.float32)

    return pl.pallas_call(
        body,
        out_shape=jax.ShapeDtypeStruct((rows, d_ff), jnp.float32),
        in_specs=[pl.BlockSpec(memory_space=pltpu.VMEM)] * 5,
        out_specs=pl.BlockSpec(memory_space=pltpu.VMEM),
        scratch_shapes=[
            pltpu.VMEM((n_tok, d_ff), jnp.bfloat16),
            pltpu.VMEM((N_DEV - 1, rows, d_ff), jnp.bfloat16),
            pltpu.SemaphoreType.DMA((N_DEV - 1,)),
            pltpu.SemaphoreType.DMA((N_DEV - 1,)),
        ],
        compiler_params=pltpu.CompilerParams(collective_id=0),
    )(x, router_W, route_idx, expert_W, shared_W)


# baseline (device time: 25940 ns/iter reference)
import jax
import jax.numpy as jnp
from jax import lax
from jax.experimental import pallas as pl
from jax.experimental.pallas import tpu as pltpu

N_DEV = 4
N_EXPERTS = 16
E_LOCAL = N_EXPERTS // N_DEV


def kernel(x, router_W, route_idx, expert_W, shared_W):
    n_tok, d_model = x.shape
    d_ff = shared_W.shape[1]
    rows = n_tok // N_DEV

    def body(x_ref, rw_ref, idx_ref, ew_ref, sw_ref, out_ref,
             acc_ref, comm_ref, send_sems, recv_sems):
        my = lax.axis_index("i")
        left = lax.rem(my + N_DEV - 1, N_DEV)
        right = lax.rem(my + 1, N_DEV)

        barrier = pltpu.get_barrier_semaphore()
        for nbr in (left, right):
            pl.semaphore_signal(
                barrier, inc=1,
                device_id=(nbr,), device_id_type=pl.DeviceIdType.MESH,
            )
        pl.semaphore_wait(barrier, 2)

        x_f32 = x_ref[:, :]
        scores = jnp.dot(x_f32, rw_ref[:, :], preferred_element_type=jnp.float32)
        scores = scores - jnp.max(scores, axis=-1, keepdims=True)
        e_scores = jnp.exp(scores)
        probs = e_scores / jnp.sum(e_scores, axis=-1, keepdims=True)

        ridx = idx_ref[:, :]
        eidx = lax.broadcasted_iota(jnp.int32, probs.shape, 1)
        x_bf = x_f32.astype(jnp.bfloat16)

        partial = jnp.zeros((n_tok, d_ff), jnp.float32)
        for e_loc in range(E_LOCAL):
            e_glob = my * E_LOCAL + e_loc
            p_e = jnp.sum(jnp.where(eidx == e_glob, probs, 0.0),
                          axis=1, keepdims=True)
            w = jnp.where(ridx == e_glob, p_e, 0.0)
            y = jnp.dot(x_bf, ew_ref[e_loc].astype(jnp.bfloat16),
                        preferred_element_type=jnp.float32)
            partial = partial + w * y
        acc_ref[:, :] = partial.astype(jnp.bfloat16)

        for s in range(N_DEV - 1):
            send_idx = lax.rem(my - (s + 1) + 2 * N_DEV, N_DEV)
            recv_idx = lax.rem(my - (s + 2) + 2 * N_DEV, N_DEV)
            rdma = pltpu.make_async_remote_copy(
                src_ref=acc_ref.at[pl.ds(send_idx * rows, rows)],
                dst_ref=comm_ref.at[s],
                send_sem=send_sems.at[s],
                recv_sem=recv_sems.at[s],
                device_id=(right,),
                device_id_type=pl.DeviceIdType.MESH,
            )
            rdma.start()
            rdma.wait()
            acc_ref[pl.ds(recv_idx * rows, rows)] = (
                acc_ref[pl.ds(recv_idx * rows, rows)] + comm_ref[s]
            )

        x_mine = x_ref[pl.ds(my * rows, rows), :].astype(jnp.bfloat16)
        shared = jnp.dot(x_mine, sw_ref[:, :].astype(jnp.bfloat16),
                         preferred_element_type=jnp.float32)
        out_ref[:, :] = shared + acc_ref[pl.ds(my * rows, rows)].astype(jnp.float32)

    return pl.pallas_call(
        body,
        out_shape=jax.ShapeDtypeStruct((rows, d_ff), jnp.float32),
        in_specs=[pl.BlockSpec(memory_space=pltpu.VMEM)] * 5,
        out_specs=pl.BlockSpec(memory_space=pltpu.VMEM),
        scratch_shapes=[
            pltpu.VMEM((n_tok, d_ff), jnp.bfloat16),
            pltpu.VMEM((N_DEV - 1, rows, d_ff), jnp.bfloat16),
            pltpu.SemaphoreType.DMA((N_DEV - 1,)),
            pltpu.SemaphoreType.DMA((N_DEV - 1,)),
        ],
        compiler_params=pltpu.CompilerParams(collective_id=0),
    )(x, router_W, route_idx, expert_W, shared_W)


# device time: 17773 ns/iter; 1.4595x vs baseline; 1.4595x over previous
import jax
import jax.numpy as jnp
from jax import lax
from jax.experimental import pallas as pl
from jax.experimental.pallas import tpu as pltpu

N_DEV = 4
N_EXPERTS = 16
E_LOCAL = N_EXPERTS // N_DEV


def kernel(x, router_W, route_idx, expert_W, shared_W):
    n_tok, d_model = x.shape
    d_ff = shared_W.shape[1]
    rows = n_tok // N_DEV

    def body(x_ref, rw_ref, idx_ref, ew_ref, sw_ref, out_ref,
             send_ref, comm_ref, send_sems, recv_sems):
        my = lax.axis_index("i")

        barrier = pltpu.get_barrier_semaphore()
        for off in (1, 2, 3):
            pl.semaphore_signal(
                barrier, inc=1,
                device_id=(lax.rem(my + off, N_DEV),),
                device_id_type=pl.DeviceIdType.MESH,
            )

        eidx = lax.broadcasted_iota(jnp.int32, (rows, N_EXPERTS), 1)

        def chunk_partial(c):
            xs = x_ref[pl.ds(c * rows, rows), :]
            scores = jnp.dot(xs, rw_ref[:, :],
                             preferred_element_type=jnp.float32)
            scores = scores - jnp.max(scores, axis=-1, keepdims=True)
            es = jnp.exp(scores)
            probs = es / jnp.sum(es, axis=-1, keepdims=True)
            ridx = idx_ref[pl.ds(c * rows, rows), :]
            xs_bf = xs.astype(jnp.bfloat16)
            part = jnp.zeros((rows, d_ff), jnp.float32)
            for e_loc in range(E_LOCAL):
                e_glob = my * E_LOCAL + e_loc
                p_e = jnp.sum(jnp.where(eidx == e_glob, probs, 0.0),
                              axis=1, keepdims=True)
                w = jnp.where(ridx == e_glob, p_e, 0.0)
                y = jnp.dot(xs_bf, ew_ref[e_loc].astype(jnp.bfloat16),
                            preferred_element_type=jnp.float32)
                part = part + w * y
            return xs_bf, part

        rdmas = []
        first = True
        for off in (2, 1, 3):
            dst = lax.rem(my + off, N_DEV)
            slot = 3 - off
            _, part = chunk_partial(dst)
            send_ref[slot] = part.astype(jnp.bfloat16)
            rdma = pltpu.make_async_remote_copy(
                src_ref=send_ref.at[slot],
                dst_ref=comm_ref.at[slot],
                send_sem=send_sems.at[slot],
                recv_sem=recv_sems.at[slot],
                device_id=(dst,),
                device_id_type=pl.DeviceIdType.MESH,
            )
            if first:
                pl.semaphore_wait(barrier, N_DEV - 1)
                first = False
            rdma.start()
            rdmas.append((rdma, slot))

        xs_bf_my, part_my = chunk_partial(my)
        shared = jnp.dot(xs_bf_my, sw_ref[:, :].astype(jnp.bfloat16),
                         preferred_element_type=jnp.float32)
        total = shared + part_my

        by_slot = {slot: rdma for rdma, slot in rdmas}
        for slot in (0, 2, 1):
            rdma = by_slot[slot]
            rdma.wait_recv()
            total = total + comm_ref[slot].astype(jnp.float32)
        out_ref[:, :] = total


        for rdma, _ in rdmas:
            rdma.wait_send()

    return pl.pallas_call(
        body,
        out_shape=jax.ShapeDtypeStruct((rows, d_ff), jnp.float32),
        in_specs=[pl.BlockSpec(memory_space=pltpu.VMEM)] * 5,
        out_specs=pl.BlockSpec(memory_space=pltpu.VMEM),
        scratch_shapes=[
            pltpu.VMEM((N_DEV - 1, rows, d_ff), jnp.bfloat16),
            pltpu.VMEM((N_DEV - 1, rows, d_ff), jnp.bfloat16),
            pltpu.SemaphoreType.DMA((N_DEV - 1,)),
            pltpu.SemaphoreType.DMA((N_DEV - 1,)),
        ],
        compiler_params=pltpu.CompilerParams(collective_id=0),
    )(x, router_W, route_idx, expert_W, shared_W)
